# Initial kernel scaffold; baseline (speedup 1.0000x reference)
import jax
import jax.numpy as jnp
from jax.experimental import pallas as pl


def kernel(input_ids, table):
    raise NotImplementedError("write your pallas kernel here")


# speedup vs baseline: 4.8037x; 4.8037x over previous
import functools

import jax
import jax.numpy as jnp
from jax import lax
from jax.experimental import pallas as pl
from jax.experimental.pallas import tpu as pltpu
from jax.experimental.pallas import tpu_sc as plsc

NC = 2
NS = 16
NW = NC * NS

GRP = 128
GRPS_PER_CHUNK = 8
CHUNK = GRP * GRPS_PER_CHUNK


@functools.partial(jax.jit, static_argnames=("n_chunks", "hidden"))
def _gather_call(ids2d, table, n_chunks, hidden):
    b_total = ids2d.shape[0] * ids2d.shape[1]
    b_per_w = b_total // NW
    rows_per_w = b_per_w // GRP
    mesh = plsc.VectorSubcoreMesh(core_axis_name="c", subcore_axis_name="s")

    @functools.partial(
        pl.kernel,
        mesh=mesh,
        out_type=jax.ShapeDtypeStruct((b_total, hidden), jnp.float32),
        scratch_types=[
            pltpu.VMEM((2, GRPS_PER_CHUNK, GRP), jnp.int32),
            pltpu.VMEM((2, CHUNK, hidden), jnp.float32),
            pltpu.SemaphoreType.DMA,
            pltpu.SemaphoreType.DMA,
            pltpu.SemaphoreType.DMA,
        ],
        compiler_params=pltpu.CompilerParams(use_tc_tiling_on_sc=False),
    )
    def k(ids_hbm, table_hbm, out_hbm, idx_v, rows_v, idx_sem, g_sem, st_sem):
        wid = lax.axis_index("s") * NC + lax.axis_index("c")
        row0 = wid * rows_per_w

        def fire_gathers(buf):
            cps = []
            for j in range(GRPS_PER_CHUNK):
                cps.append(
                    pltpu.async_copy(
                        table_hbm.at[idx_v.at[buf, j]],
                        rows_v.at[buf, pl.ds(j * GRP, GRP)],
                        g_sem,
                    )
                )
            return cps

        def body(g, _):
            buf = lax.rem(g, 2)
            pltpu.async_copy(
                ids_hbm.at[pl.ds(row0 + g * GRPS_PER_CHUNK, GRPS_PER_CHUNK)],
                idx_v.at[buf],
                idx_sem,
            ).wait()
            for cp in fire_gathers(buf):
                cp.wait()
            st = pltpu.async_copy(
                rows_v.at[buf],
                out_hbm.at[pl.ds((row0 + g * GRPS_PER_CHUNK) * GRP, CHUNK)],
                st_sem,
            )
            st.wait()
            return 0

        lax.fori_loop(0, n_chunks, body, 0, unroll=False)

    return k(ids2d, table)


def kernel(input_ids, table):
    b, t = input_ids.shape
    hidden = table.shape[1]
    b_total = b * t
    assert b_total % (NW * CHUNK) == 0
    n_chunks = b_total // (NW * CHUNK)
    ids2d = input_ids.reshape(b_total // GRP, GRP)
    out = _gather_call(ids2d, table, n_chunks, hidden)
    return out.reshape(b, t, hidden)


# speedup vs baseline: 5.0246x; 1.0460x over previous
import functools

import jax
import jax.numpy as jnp
from jax import lax
from jax.experimental import pallas as pl
from jax.experimental.pallas import tpu as pltpu
from jax.experimental.pallas import tpu_sc as plsc

NC = 2
NS = 16
NW = NC * NS

GRP = 128
GRPS_PER_CHUNK = 8
CHUNK = GRP * GRPS_PER_CHUNK


@functools.partial(jax.jit, static_argnames=("n_chunks", "hidden"))
def _gather_call(ids2d, table, n_chunks, hidden):
    total_rows = ids2d.shape[0]
    b_total = total_rows * GRP
    rows_per_w = total_rows // NW
    mesh = plsc.VectorSubcoreMesh(core_axis_name="c", subcore_axis_name="s")

    @functools.partial(
        pl.kernel,
        mesh=mesh,
        out_type=jax.ShapeDtypeStruct((b_total, hidden), jnp.float32),
        scratch_types=[
            pltpu.VMEM((2, GRPS_PER_CHUNK, GRP), jnp.int32),
            pltpu.VMEM((2, CHUNK, hidden), jnp.float32),
            pltpu.SemaphoreType.DMA,
            pltpu.SemaphoreType.DMA,
            pltpu.SemaphoreType.DMA,
            pltpu.SemaphoreType.DMA,
        ],
        compiler_params=pltpu.CompilerParams(use_tc_tiling_on_sc=False),
    )
    def k(ids_hbm, table_hbm, out_hbm, idx_v, rows_v, idx_sem, g_sem,
          st_sem0, st_sem1):
        wid = lax.axis_index("s") * NC + lax.axis_index("c")
        row0 = wid * rows_per_w

        def idx_copy(g, buf):
            row = jnp.minimum(row0 + g * GRPS_PER_CHUNK,
                              total_rows - GRPS_PER_CHUNK)
            return pltpu.make_async_copy(
                ids_hbm.at[pl.ds(row, GRPS_PER_CHUNK)], idx_v.at[buf],
                idx_sem)

        def gather_copies(buf):
            return [
                pltpu.make_async_copy(
                    table_hbm.at[idx_v.at[buf, j]],
                    rows_v.at[buf, pl.ds(j * GRP, GRP)],
                    g_sem)
                for j in range(GRPS_PER_CHUNK)
            ]

        def store_copy(g, buf, sem):
            off = (row0 + g * GRPS_PER_CHUNK) * GRP
            return pltpu.make_async_copy(
                rows_v.at[buf], out_hbm.at[pl.ds(off, CHUNK)], sem)

        def run_chunk(g, buf, st_sem, prefetch_g, prefetch_buf):
            idx_copy(g, buf).wait()
            for c in gather_copies(buf):
                c.start()
            idx_copy(prefetch_g, prefetch_buf).start()
            for c in gather_copies(buf):
                c.wait()
            store_copy(g, buf, st_sem).start()

        idx_copy(0, 0).start()
        run_chunk(0, 0, st_sem0, 1, 1)
        run_chunk(1, 1, st_sem1, 2, 0)

        def body(g2, _):
            g = 2 * g2
            store_copy(g - 2, 0, st_sem0).wait()
            run_chunk(g, 0, st_sem0, g + 1, 1)
            store_copy(g - 1, 1, st_sem1).wait()
            run_chunk(g + 1, 1, st_sem1, g + 2, 0)
            return 0

        lax.fori_loop(1, n_chunks // 2, body, 0, unroll=False)

        idx_copy(n_chunks, 0).wait()
        store_copy(n_chunks - 2, 0, st_sem0).wait()
        store_copy(n_chunks - 1, 1, st_sem1).wait()

    return k(ids2d, table)


def kernel(input_ids, table):
    b, t = input_ids.shape
    hidden = table.shape[1]
    b_total = b * t
    assert b_total % (NW * CHUNK) == 0 and (b_total // (NW * CHUNK)) % 2 == 0
    n_chunks = b_total // (NW * CHUNK)
    ids2d = input_ids.reshape(b_total // GRP, GRP)
    out = _gather_call(ids2d, table, n_chunks, hidden)
    return out.reshape(b, t, hidden)
